# Initial kernel scaffold; baseline (speedup 1.0000x reference)
import jax
import jax.numpy as jnp
from jax.experimental import pallas as pl


def kernel(logits, edge_index, edge_values):
    raise NotImplementedError("write your pallas kernel here")


# speedup vs baseline: 2.3090x; 2.3090x over previous
import functools

import jax
import jax.numpy as jnp
from jax import lax
from jax.experimental import pallas as pl
from jax.experimental.pallas import tpu as pltpu
from jax.experimental.pallas import tpu_sc as plsc

N = 128
V = 10000
E = 320000
LANES = 16
ROWS_PER_TILE = 4
CHUNK = 4000


def _sc_body(flat_hbm, col_hbm, row_hbm, vv_hbm, out_hbm,
             x_v, acc_v, col_v, row_v, vv_v):
    nc = plsc.get_sparse_core_info().num_cores
    wid = lax.axis_index("s") * nc + lax.axis_index("c")
    base = wid * ROWS_PER_TILE * V

    pltpu.sync_copy(flat_hbm.at[pl.ds(base, ROWS_PER_TILE * V)], x_v)
    pltpu.sync_copy(flat_hbm.at[pl.ds(base, ROWS_PER_TILE * V)], acc_v)

    joffs = [jnp.full((LANES,), j * V, jnp.int32) for j in range(ROWS_PER_TILE)]

    @pl.loop(0, E, step=CHUNK)
    def _chunk(e0):
        pltpu.sync_copy(col_hbm.at[pl.ds(e0, CHUNK)], col_v)
        pltpu.sync_copy(row_hbm.at[pl.ds(e0, CHUNK)], row_v)
        pltpu.sync_copy(vv_hbm.at[pl.ds(e0, CHUNK)], vv_v)

        @pl.loop(0, CHUNK, step=LANES)
        def _grp(i):
            c = col_v[pl.ds(i, LANES)]
            r = row_v[pl.ds(i, LANES)]
            w = vv_v[pl.ds(i, LANES)]
            for j in range(ROWS_PER_TILE):
                g = plsc.load_gather(x_v, [c + joffs[j]])
                plsc.addupdate_scatter(acc_v, [r + joffs[j]], g * w)

    pltpu.sync_copy(acc_v, out_hbm.at[pl.ds(base, ROWS_PER_TILE * V)])


@jax.jit
def _structured_logits_sc(flat, col, row, vv):
    flat = flat.reshape(-1)
    mesh = plsc.VectorSubcoreMesh(core_axis_name="c", subcore_axis_name="s")
    return pl.kernel(
        _sc_body,
        out_type=jax.ShapeDtypeStruct((N * V,), jnp.float32),
        mesh=mesh,
        compiler_params=pltpu.CompilerParams(needs_layout_passes=False),
        scratch_types=[
            pltpu.VMEM((ROWS_PER_TILE * V,), jnp.float32),
            pltpu.VMEM((ROWS_PER_TILE * V,), jnp.float32),
            pltpu.VMEM((CHUNK,), jnp.int32),
            pltpu.VMEM((CHUNK,), jnp.int32),
            pltpu.VMEM((CHUNK,), jnp.float32),
        ],
    )(flat, col, row, vv)


def kernel(logits, edge_index, edge_values):
    old_shape = logits.shape
    flat = logits.reshape(-1, old_shape[-1])
    row = edge_index[0]
    col = edge_index[1]
    out = _structured_logits_sc(flat, col, row, edge_values)
    return out.reshape(old_shape)


# speedup vs baseline: 7.3419x; 3.1796x over previous
import jax
import jax.numpy as jnp
from jax import lax
from jax.experimental import pallas as pl
from jax.experimental.pallas import tpu as pltpu
from jax.experimental.pallas import tpu_sc as plsc

N = 128
V = 10000
E = 320000
LANES = 16
ROWS_PER_TILE = 4
CHUNK = 4000
NCHUNKS = E // CHUNK


def _sc_body(flat_hbm, col_hbm, row_hbm, vv_hbm, out_hbm,
             x_v, acc_v, col_b0, row_b0, vv_b0, col_b1, row_b1, vv_b1,
             sem0, sem1, xsem):
    nc = plsc.get_sparse_core_info().num_cores
    wid = lax.axis_index("s") * nc + lax.axis_index("c")
    base = wid * ROWS_PER_TILE * V

    bufs = ((col_b0, row_b0, vv_b0, sem0), (col_b1, row_b1, vv_b1, sem1))

    def start(slot, e0):
        col_v, row_v, vv_v, sem = bufs[slot]
        pltpu.async_copy(col_hbm.at[pl.ds(e0, CHUNK)], col_v, sem)
        pltpu.async_copy(row_hbm.at[pl.ds(e0, CHUNK)], row_v, sem)
        pltpu.async_copy(vv_hbm.at[pl.ds(e0, CHUNK)], vv_v, sem)

    def wait(slot):
        col_v, row_v, vv_v, sem = bufs[slot]
        pltpu.make_async_copy(col_hbm.at[pl.ds(0, CHUNK)], col_v, sem).wait()
        pltpu.make_async_copy(row_hbm.at[pl.ds(0, CHUNK)], row_v, sem).wait()
        pltpu.make_async_copy(vv_hbm.at[pl.ds(0, CHUNK)], vv_v, sem).wait()

    start(0, 0)
    start(1, CHUNK)
    pltpu.async_copy(flat_hbm.at[pl.ds(base, ROWS_PER_TILE * V)], x_v, xsem)
    pltpu.async_copy(flat_hbm.at[pl.ds(base, ROWS_PER_TILE * V)], acc_v, xsem)
    pltpu.make_async_copy(flat_hbm.at[pl.ds(0, ROWS_PER_TILE * V)], x_v, xsem).wait()
    pltpu.make_async_copy(flat_hbm.at[pl.ds(0, ROWS_PER_TILE * V)], acc_v, xsem).wait()

    joffs = [jnp.full((LANES,), j * V, jnp.int32) for j in range(ROWS_PER_TILE)]

    def process(slot):
        col_v, row_v, vv_v, _ = bufs[slot]

        @plsc.parallel_loop(0, CHUNK, LANES, unroll=4)
        def _grp(i):
            c = col_v[pl.ds(i, LANES)]
            r = row_v[pl.ds(i, LANES)]
            w = vv_v[pl.ds(i, LANES)]
            for j in range(ROWS_PER_TILE):
                g = plsc.load_gather(x_v, [c + joffs[j]])
                plsc.addupdate_scatter(acc_v, [r + joffs[j]], g * w)

    @pl.loop(0, NCHUNKS, step=2)
    def _pair(g):
        wait(0)
        process(0)

        @pl.when(g + 2 < NCHUNKS)
        def _():
            start(0, (g + 2) * CHUNK)

        wait(1)
        process(1)

        @pl.when(g + 3 < NCHUNKS)
        def _():
            start(1, (g + 3) * CHUNK)

    pltpu.sync_copy(acc_v, out_hbm.at[pl.ds(base, ROWS_PER_TILE * V)])


@jax.jit
def _structured_logits_sc(flat, col, row, vv):
    flat = flat.reshape(-1)
    mesh = plsc.VectorSubcoreMesh(core_axis_name="c", subcore_axis_name="s")
    return pl.kernel(
        _sc_body,
        out_type=jax.ShapeDtypeStruct((N * V,), jnp.float32),
        mesh=mesh,
        compiler_params=pltpu.CompilerParams(needs_layout_passes=False),
        scratch_types=[
            pltpu.VMEM((ROWS_PER_TILE * V,), jnp.float32),
            pltpu.VMEM((ROWS_PER_TILE * V,), jnp.float32),
            pltpu.VMEM((CHUNK,), jnp.int32),
            pltpu.VMEM((CHUNK,), jnp.int32),
            pltpu.VMEM((CHUNK,), jnp.float32),
            pltpu.VMEM((CHUNK,), jnp.int32),
            pltpu.VMEM((CHUNK,), jnp.int32),
            pltpu.VMEM((CHUNK,), jnp.float32),
            pltpu.SemaphoreType.DMA,
            pltpu.SemaphoreType.DMA,
            pltpu.SemaphoreType.DMA,
        ],
    )(flat, col, row, vv)


def kernel(logits, edge_index, edge_values):
    old_shape = logits.shape
    flat = logits.reshape(-1, old_shape[-1])
    row = edge_index[0]
    col = edge_index[1]
    out = _structured_logits_sc(flat, col, row, edge_values)
    return out.reshape(old_shape)
